# Initial kernel scaffold; baseline (speedup 1.0000x reference)
import jax
import jax.numpy as jnp
from jax.experimental import pallas as pl


def kernel(modality_features_0, modality_features_1, gate_W, W1, b1, W2, b2):
    raise NotImplementedError("write your pallas kernel here")


# speedup vs baseline: 1.1507x; 1.1507x over previous
import functools

import jax
import jax.numpy as jnp
from jax.experimental import pallas as pl
from jax.experimental.pallas import tpu as pltpu

B, T, D = 1, 2048, 768
M = 2
E, K, H = 8, 2, 3072
N = B * M * T
HC = 512
NHC = H // HC


def _moe_dense_kernel(x_ref, gwt_ref, w1_ref, b1_ref, w2_ref, b2_ref,
                      out_ref, xbf_scr, w_scr):
    e = pl.program_id(0)
    hc = pl.program_id(1)

    @pl.when(jnp.logical_and(e == 0, hc == 0))
    def _init():
        x = x_ref[...]
        xbf_scr[...] = x.astype(jnp.bfloat16)
        logits = jax.lax.dot_general(
            x, gwt_ref[...], (((1,), (0,)), ((), ())),
            preferred_element_type=jnp.float32)
        i1 = jnp.argmax(logits, axis=-1)
        m1 = jnp.max(logits, axis=-1)
        eids = jax.lax.broadcasted_iota(jnp.int32, logits.shape, 1)
        masked = jnp.where(eids == i1[:, None], -jnp.inf, logits)
        i2 = jnp.argmax(masked, axis=-1)
        m2 = jnp.max(masked, axis=-1)
        z = jnp.exp(m2 - m1)
        p1 = 1.0 / (1.0 + z)
        p2 = 1.0 - p1
        w = (jnp.where(eids == i1[:, None], p1[:, None], 0.0)
             + jnp.where(eids == i2[:, None], p2[:, None], 0.0))
        w_scr[...] = w
        out_ref[...] = jnp.zeros_like(out_ref)

    w_all = w_scr[...]
    sel = jax.lax.broadcasted_iota(jnp.int32, w_all.shape, 1) == e
    w_col = jnp.sum(jnp.where(sel, w_all, 0.0), axis=-1, keepdims=True)
    xbf = xbf_scr[...]
    w1 = w1_ref[0].astype(jnp.bfloat16)
    h = jax.lax.dot_general(xbf, w1, (((1,), (0,)), ((), ())),
                            preferred_element_type=jnp.float32)
    h = jnp.maximum(h + b1_ref[0], 0.0)
    h = (h * w_col).astype(jnp.bfloat16)
    w2 = w2_ref[0].astype(jnp.bfloat16)
    y = jax.lax.dot_general(h, w2, (((1,), (0,)), ((), ())),
                            preferred_element_type=jnp.float32)
    out_ref[...] += y

    @pl.when(hc == 0)
    def _bias2():
        out_ref[...] += w_col * b2_ref[0]


@functools.partial(jax.jit, static_argnames=())
def kernel(modality_features_0, modality_features_1, gate_W, W1, b1, W2, b2):
    x_flat = jnp.concatenate(
        [modality_features_0.reshape(T, D), modality_features_1.reshape(T, D)],
        axis=0)
    gwt = gate_W.T

    out = pl.pallas_call(
        _moe_dense_kernel,
        grid=(E, NHC),
        in_specs=[
            pl.BlockSpec((N, D), lambda e, hc: (0, 0)),
            pl.BlockSpec((D, E), lambda e, hc: (0, 0)),
            pl.BlockSpec((1, D, HC), lambda e, hc: (e, 0, hc)),
            pl.BlockSpec((1, 1, HC), lambda e, hc: (e, 0, hc)),
            pl.BlockSpec((1, HC, D), lambda e, hc: (e, hc, 0)),
            pl.BlockSpec((1, 1, D), lambda e, hc: (e, 0, 0)),
        ],
        out_specs=pl.BlockSpec((N, D), lambda e, hc: (0, 0)),
        out_shape=jax.ShapeDtypeStruct((N, D), jnp.float32),
        scratch_shapes=[
            pltpu.VMEM((N, D), jnp.bfloat16),
            pltpu.VMEM((N, E), jnp.float32),
        ],
        compiler_params=pltpu.CompilerParams(
            dimension_semantics=("arbitrary", "arbitrary")),
    )(x_flat, gwt, W1, b1.reshape(E, 1, H), W2, b2.reshape(E, 1, D))

    fused = 0.5 * (out[:T] + out[T:])
    return fused.reshape(B, T, D)


# speedup vs baseline: 1.5142x; 1.3159x over previous
import functools

import jax
from jax import lax
import jax.numpy as jnp
from jax.experimental import pallas as pl
from jax.experimental.pallas import tpu as pltpu
from jax.experimental.pallas import tpu_sc as plsc

B, T, D = 1, 2048, 768
M = 2
E, K, H = 8, 2, 3072
N = B * M * T
NS = N * K
BT = 512
NB = NS // BT
P_MAX = NB + E
NW = 32


def _router_kernel(x_ref, gwt_ref, idx_ref, p_ref):
    logits = lax.dot_general(x_ref[...], gwt_ref[...], (((1,), (0,)), ((), ())),
                             preferred_element_type=jnp.float32)
    eids = lax.broadcasted_iota(jnp.int32, logits.shape, 1)
    i1 = jnp.argmax(logits, axis=-1)
    m1 = jnp.max(logits, axis=-1)
    masked = jnp.where(eids == i1[:, None], -jnp.inf, logits)
    i2 = jnp.argmax(masked, axis=-1)
    m2 = jnp.max(masked, axis=-1)
    z = jnp.exp(m2 - m1)
    p1 = 1.0 / (1.0 + z)
    idx_ref[...] = jnp.concatenate(
        [i1[:, None], i2[:, None]], axis=1).astype(jnp.int32)
    p_ref[...] = jnp.concatenate(
        [p1[:, None], (1.0 - p1)[:, None]], axis=1)


def _run_router(x_flat, gate_W):
    return pl.pallas_call(
        _router_kernel,
        in_specs=[pl.BlockSpec((N, D), lambda: (0, 0)),
                  pl.BlockSpec((D, E), lambda: (0, 0))],
        out_specs=[pl.BlockSpec((N, K), lambda: (0, 0)),
                   pl.BlockSpec((N, K), lambda: (0, 0))],
        out_shape=[jax.ShapeDtypeStruct((N, K), jnp.int32),
                   jax.ShapeDtypeStruct((N, K), jnp.float32)],
    )(x_flat, gate_W.T)


def _sc_gather_rows(table, idx):
    ntot = idx.shape[0]
    d = table.shape[1]
    per_w = ntot // NW
    ch = 64
    nchunk = per_w // ch
    mesh = plsc.VectorSubcoreMesh(core_axis_name="c", subcore_axis_name="s")

    @functools.partial(
        pl.kernel, mesh=mesh,
        out_type=jax.ShapeDtypeStruct((ntot, d), table.dtype),
        scratch_types=[
            pltpu.VMEM((per_w,), jnp.int32),
            pltpu.VMEM((ch, d), table.dtype),
            pltpu.VMEM((ch, d), table.dtype),
            pltpu.SemaphoreType.DMA,
            pltpu.SemaphoreType.DMA,
        ],
    )
    def k(table_hbm, idx_hbm, out_hbm, idx_v, buf0, buf1, sem0, sem1):
        wid = lax.axis_index("s") * 2 + lax.axis_index("c")
        base = wid * per_w
        pltpu.sync_copy(idx_hbm.at[pl.ds(base, per_w)], idx_v)
        bufs = (buf0, buf1)
        sems = (sem0, sem1)
        handles = [None] * nchunk
        handles[0] = pltpu.async_copy(
            table_hbm.at[idx_v.at[pl.ds(0, ch)]], bufs[0], sems[0])
        for c in range(nchunk):
            if c + 1 < nchunk:
                handles[c + 1] = pltpu.async_copy(
                    table_hbm.at[idx_v.at[pl.ds((c + 1) * ch, ch)]],
                    bufs[(c + 1) % 2], sems[(c + 1) % 2])
            handles[c].wait()
            pltpu.sync_copy(bufs[c % 2], out_hbm.at[pl.ds(base + c * ch, ch)])

    return k(table, idx)


def _ffn_kernel(pb_ref, pe_ref, pf_ref, x_ref, es_ref, ps_ref,
                w1_ref, b1_ref, w2_ref, b2_ref, out_ref):
    p = pl.program_id(0)
    flags = pf_ref[p]
    valid = (flags & 1) == 1
    first = (flags & 2) == 2

    @pl.when(first)
    def _init():
        out_ref[...] = jnp.zeros_like(out_ref)

    @pl.when(valid)
    def _body():
        e = pe_ref[p]
        mask = es_ref[...] == e
        w_col = jnp.where(mask, ps_ref[...], 0.0) * 0.5
        x = x_ref[...].astype(jnp.bfloat16)
        h = lax.dot_general(x, w1_ref[0].astype(jnp.bfloat16),
                            (((1,), (0,)), ((), ())),
                            preferred_element_type=jnp.float32)
        h = jnp.maximum(h + b1_ref[0], 0.0)
        h = (h * w_col).astype(jnp.bfloat16)
        y = lax.dot_general(h, w2_ref[0].astype(jnp.bfloat16),
                            (((1,), (0,)), ((), ())),
                            preferred_element_type=jnp.float32)
        out_ref[...] += y + w_col * b2_ref[0]


def _run_ffn(x_sorted, e_sorted, p_sorted, W1, b1, W2, b2, pb, pe, pf):
    grid_spec = pltpu.PrefetchScalarGridSpec(
        num_scalar_prefetch=3,
        grid=(P_MAX,),
        in_specs=[
            pl.BlockSpec((BT, D), lambda p, pb, pe, pf: (pb[p], 0)),
            pl.BlockSpec((BT, 1), lambda p, pb, pe, pf: (pb[p], 0)),
            pl.BlockSpec((BT, 1), lambda p, pb, pe, pf: (pb[p], 0)),
            pl.BlockSpec((1, D, H), lambda p, pb, pe, pf: (pe[p], 0, 0)),
            pl.BlockSpec((1, 1, H), lambda p, pb, pe, pf: (pe[p], 0, 0)),
            pl.BlockSpec((1, H, D), lambda p, pb, pe, pf: (pe[p], 0, 0)),
            pl.BlockSpec((1, 1, D), lambda p, pb, pe, pf: (pe[p], 0, 0)),
        ],
        out_specs=pl.BlockSpec((BT, D), lambda p, pb, pe, pf: (pb[p], 0)),
    )
    return pl.pallas_call(
        _ffn_kernel,
        grid_spec=grid_spec,
        out_shape=jax.ShapeDtypeStruct((NS, D), jnp.float32),
        compiler_params=pltpu.CompilerParams(
            dimension_semantics=("arbitrary",)),
    )(pb, pe, pf, x_sorted, e_sorted, p_sorted,
      W1, b1.reshape(E, 1, H), W2, b2.reshape(E, 1, D))


BTC = 512
NBC = T // BTC


def _sum4_kernel(a_ref, b_ref, c_ref, d_ref, out_ref):
    out_ref[...] = a_ref[...] + b_ref[...] + c_ref[...] + d_ref[...]


def _run_sum4(y_perm):
    specs = [pl.BlockSpec((BTC, D), (lambda j: lambda i: (j * NBC + i, 0))(j))
             for j in range(4)]
    return pl.pallas_call(
        _sum4_kernel,
        grid=(NBC,),
        in_specs=specs,
        out_specs=pl.BlockSpec((BTC, D), lambda i: (i, 0)),
        out_shape=jax.ShapeDtypeStruct((T, D), jnp.float32),
    )(y_perm, y_perm, y_perm, y_perm)


def _build_worklist(e_flat):
    sizes = jnp.bincount(e_flat, length=E)
    ends = jnp.cumsum(sizes)
    starts = ends - sizes
    first_blk = starts // BT
    last_blk = jnp.where(sizes > 0, (ends - 1) // BT, first_blk)
    nb = jnp.where(sizes > 0, last_blk - first_blk + 1, 0)
    cum_nb = jnp.cumsum(nb)
    pair_start = cum_nb - nb
    total = cum_nb[-1]
    parange = jnp.arange(P_MAX, dtype=jnp.int32)
    pe_raw = jnp.searchsorted(cum_nb, parange, side="right").astype(jnp.int32)
    pe_c = jnp.minimum(pe_raw, E - 1)
    pb_raw = (first_blk[pe_c] + (parange - pair_start[pe_c])).astype(jnp.int32)
    valid = parange < total
    last_e = jnp.take(pe_c, total - 1)
    last_b = jnp.take(pb_raw, total - 1)
    pe = jnp.where(valid, pe_c, last_e).astype(jnp.int32)
    pb = jnp.where(valid, pb_raw, last_b).astype(jnp.int32)
    prev_pb = jnp.concatenate([jnp.full((1,), -1, jnp.int32), pb[:-1]])
    first = valid & (pb != prev_pb)
    pf = (valid.astype(jnp.int32) + 2 * first.astype(jnp.int32))
    return pb, pe, pf


def kernel(modality_features_0, modality_features_1, gate_W, W1, b1, W2, b2):
    x_flat = jnp.concatenate(
        [modality_features_0.reshape(T, D), modality_features_1.reshape(T, D)],
        axis=0)

    top_idx, top_p = _run_router(x_flat, gate_W)

    e_flat = top_idx.reshape(NS)
    p_flat = top_p.reshape(NS)
    perm = jnp.argsort(e_flat)
    tok_sorted = (perm // K).astype(jnp.int32)
    e_sorted = e_flat[perm].astype(jnp.int32)
    p_sorted = p_flat[perm]
    pb, pe, pf = _build_worklist(e_flat)

    pos = jnp.zeros((NS,), jnp.int32).at[perm].set(
        jnp.arange(NS, dtype=jnp.int32))
    pos2 = pos.reshape(N, K)
    inv_flat = jnp.concatenate(
        [pos2[:T, 0], pos2[:T, 1], pos2[T:, 0], pos2[T:, 1]], axis=0)

    x_sorted = _sc_gather_rows(x_flat, tok_sorted)
    y_sorted = _run_ffn(x_sorted, e_sorted.reshape(NS, 1),
                        p_sorted.reshape(NS, 1), W1, b1, W2, b2, pb, pe, pf)
    y_perm = _sc_gather_rows(y_sorted, inv_flat)
    fused = _run_sum4(y_perm)
    return fused.reshape(B, T, D)
